# baseline (speedup 1.0000x reference)
import jax
import jax.numpy as jnp
from jax.experimental import pallas as pl

_B, _N = 128, 106
_IN_H, _IN_W = 512, 512
_UPSCALE = 4
_STRIDE = 4
_OUT_H = int(_IN_H * _UPSCALE / _STRIDE)
_OUT_W = int(_IN_W * _UPSCALE / _STRIDE)
_SIGMA = 2.0
_RADIUS = int(_SIGMA * 3)


def _gauss_block(lmks_ref, vx_ref, vy_ref):
    l = lmks_ref[0]
    scaled = l * (_UPSCALE / _STRIDE)
    xi = scaled[:, 0:1].astype(jnp.int32)
    yi = scaled[:, 1:2].astype(jnp.int32)
    ulx, uly = xi - _RADIUS, yi - _RADIUS
    brx, bry = xi + _RADIUS + 1, yi + _RADIUS + 1

    def in_img(px, py):
        return jnp.logical_not((px < 0) | (px > _OUT_W) | (py < 0) | (py > _OUT_H))

    valid = in_img(ulx, uly) | in_img(brx, bry)
    neg_inv = -1.0 / (2.0 * _SIGMA * _SIGMA)

    def emit(ci, out_ref, size):
        w = jax.lax.broadcasted_iota(jnp.int32, (_N, size), 1)
        d = w - ci
        m = (d >= -_RADIUS) & (d <= _RADIUS) & valid
        df = d.astype(jnp.float32)
        out_ref[0] = jnp.where(m, jnp.exp(df * df * neg_inv), 0.0)

    emit(xi, vx_ref, _OUT_W)
    emit(yi, vy_ref, _OUT_H)


def kernel(lmks):
    out_shape = [
        jax.ShapeDtypeStruct((_B, _N, _OUT_W), jnp.float32),
        jax.ShapeDtypeStruct((_B, _N, _OUT_H), jnp.float32),
    ]
    vx, vy = pl.pallas_call(
        _gauss_block,
        grid=(_B,),
        in_specs=[pl.BlockSpec((1, _N, 2), lambda b: (b, 0, 0))],
        out_specs=[
            pl.BlockSpec((1, _N, _OUT_W), lambda b: (b, 0, 0)),
            pl.BlockSpec((1, _N, _OUT_H), lambda b: (b, 0, 0)),
        ],
        out_shape=out_shape,
    )(lmks)
    return vx, vy


# speedup vs baseline: 1.4487x; 1.4487x over previous
import jax
import jax.numpy as jnp
from jax.experimental import pallas as pl

_B, _N = 128, 106
_IN_H, _IN_W = 512, 512
_UPSCALE = 4
_STRIDE = 4
_OUT_H = int(_IN_H * _UPSCALE / _STRIDE)
_OUT_W = int(_IN_W * _UPSCALE / _STRIDE)
_SIGMA = 2.0
_RADIUS = int(_SIGMA * 3)


_BB = 8


def _gauss_block(lmks_ref, vx_ref, vy_ref):
    l = lmks_ref[...]
    scaled = l * (_UPSCALE / _STRIDE)
    xi = scaled[:, :, 0:1].astype(jnp.int32)
    yi = scaled[:, :, 1:2].astype(jnp.int32)
    ulx, uly = xi - _RADIUS, yi - _RADIUS
    brx, bry = xi + _RADIUS + 1, yi + _RADIUS + 1

    def in_img(px, py):
        return jnp.logical_not((px < 0) | (px > _OUT_W) | (py < 0) | (py > _OUT_H))

    valid = in_img(ulx, uly) | in_img(brx, bry)
    neg_inv = -1.0 / (2.0 * _SIGMA * _SIGMA)

    def emit(ci, out_ref, size):
        w = jax.lax.broadcasted_iota(jnp.int32, (_BB, _N, size), 2)
        d = w - ci
        m = (d >= -_RADIUS) & (d <= _RADIUS) & valid
        df = d.astype(jnp.float32)
        out_ref[...] = jnp.where(m, jnp.exp(df * df * neg_inv), 0.0)

    emit(xi, vx_ref, _OUT_W)
    emit(yi, vy_ref, _OUT_H)


def kernel(lmks):
    out_shape = [
        jax.ShapeDtypeStruct((_B, _N, _OUT_W), jnp.float32),
        jax.ShapeDtypeStruct((_B, _N, _OUT_H), jnp.float32),
    ]
    vx, vy = pl.pallas_call(
        _gauss_block,
        grid=(_B // _BB,),
        in_specs=[pl.BlockSpec((_BB, _N, 2), lambda b: (b, 0, 0))],
        out_specs=[
            pl.BlockSpec((_BB, _N, _OUT_W), lambda b: (b, 0, 0)),
            pl.BlockSpec((_BB, _N, _OUT_H), lambda b: (b, 0, 0)),
        ],
        out_shape=out_shape,
    )(lmks)
    return vx, vy
